# baseline (speedup 1.0000x reference)
import functools

import jax
import jax.numpy as jnp
from jax.experimental import pallas as pl
from jax.experimental.pallas import tpu as pltpu

N = 4096
D = 512
K = 8

R_TILE = 256
C_TILE = 4096
R_SUB = 32

BIG_F32 = 3.0e38


def _extract_top8(keys, vals):
    ks = []
    vs = []
    for _ in range(K):
        m = jnp.min(keys, axis=1, keepdims=True)
        loc = keys == m
        ks.append(m)
        vs.append(jnp.sum(jnp.where(loc, vals, 0.0), axis=1, keepdims=True))
        keys = jnp.where(loc, BIG_F32, keys)
    return jnp.concatenate(ks, axis=1), jnp.concatenate(vs, axis=1)


def _topk_body(nc, xr_ref, xdr_ref, xc_ref, a_ref, b_ref):
    j = pl.program_id(1)
    xc = xc_ref[...]

    dims = (((1,), (1,)), ((), ()))
    sq_c = jax.lax.dot_general(jnp.ones((1, D), jnp.float32), xc * xc, dims,
                               preferred_element_type=jnp.float32)
    p = jax.lax.dot_general(xr_ref[...], xc, dims,
                            preferred_element_type=jnp.float32)
    q = jax.lax.dot_general(xdr_ref[...], xc, dims,
                            preferred_element_type=jnp.float32)

    is_first = j == 0
    is_last = j == nc - 1

    for blk in range(R_TILE // R_SUB):
        i0 = blk * R_SUB
        kb = sq_c - 2.0 * p[i0:i0 + R_SUB, :]
        vb = sq_c - 2.0 * q[i0:i0 + R_SUB, :]
        tk, tv = _extract_top8(kb, vb)

        if nc == 1:
            nk, nv = tk, tv
        else:
            prev_k = jnp.where(is_first, BIG_F32, a_ref[i0:i0 + R_SUB, :])
            prev_v = jnp.where(is_first, 0.0, b_ref[i0:i0 + R_SUB, :])
            ck = jnp.concatenate([prev_k, tk], axis=1)
            cv = jnp.concatenate([prev_v, tv], axis=1)
            nk, nv = _extract_top8(ck, cv)

        fin_k = nk + jnp.sum(jnp.square(xr_ref[i0:i0 + R_SUB, :]),
                             axis=1, keepdims=True)
        fin_v = nv + jnp.sum(jnp.square(xdr_ref[i0:i0 + R_SUB, :]),
                             axis=1, keepdims=True)
        a_ref[i0:i0 + R_SUB, :] = jnp.where(is_last, fin_k, nk)
        b_ref[i0:i0 + R_SUB, :] = jnp.where(is_last, fin_v, nv)


def _finalize_body(a_ref, b_ref, o_ref):
    a = a_ref[...]
    b = b_ref[...]
    t = jnp.mean(a) + 1e-9
    o_ref[0, 0] = jnp.mean(jnp.exp(-a / t) * b)


@jax.jit
def kernel(X, X_dot):
    nr = N // R_TILE
    nc = N // C_TILE
    a, b = pl.pallas_call(
        functools.partial(_topk_body, nc),
        grid=(nr, nc),
        in_specs=[
            pl.BlockSpec((R_TILE, D), lambda i, j: (i, 0)),
            pl.BlockSpec((R_TILE, D), lambda i, j: (i, 0)),
            pl.BlockSpec((C_TILE, D), lambda i, j: (j, 0)),
        ],
        out_specs=[
            pl.BlockSpec((R_TILE, K), lambda i, j: (i, 0)),
            pl.BlockSpec((R_TILE, K), lambda i, j: (i, 0)),
        ],
        out_shape=[
            jax.ShapeDtypeStruct((N, K), jnp.float32),
            jax.ShapeDtypeStruct((N, K), jnp.float32),
        ],
        compiler_params=pltpu.CompilerParams(
            dimension_semantics=("parallel", "arbitrary"),
        ),
    )(X, X_dot, X)

    out = pl.pallas_call(
        _finalize_body,
        out_specs=pl.BlockSpec(memory_space=pltpu.SMEM),
        out_shape=jax.ShapeDtypeStruct((1, 1), jnp.float32),
    )(a, b)
    return out[0, 0]


# speedup vs baseline: 1.0784x; 1.0784x over previous
import functools

import jax
import jax.numpy as jnp
from jax.experimental import pallas as pl
from jax.experimental.pallas import tpu as pltpu

N = 4096
D = 512
K = 8

R_TILE = 256
C_TILE = 4096
R_SUB = 16

BIG_F32 = 3.0e38


def _extract_top8(keys, vals):
    ks = []
    vs = []
    for _ in range(K):
        m = jnp.min(keys, axis=1, keepdims=True)
        loc = keys == m
        ks.append(m)
        vs.append(jnp.sum(jnp.where(loc, vals, 0.0), axis=1, keepdims=True))
        keys = jnp.where(loc, BIG_F32, keys)
    return jnp.concatenate(ks, axis=1), jnp.concatenate(vs, axis=1)


def _topk_body(nc, xr_ref, xdr_ref, xc_ref, a_ref, b_ref, sq_ref):
    j = pl.program_id(1)
    xc = xc_ref[...]

    dims = (((1,), (1,)), ((), ()))

    @pl.when(jnp.logical_and(pl.program_id(0) == 0, j == 0))
    def _():
        sq_ref[...] = jax.lax.dot_general(
            jnp.ones((1, D), jnp.float32), xc * xc, dims,
            preferred_element_type=jnp.float32)

    sq_c = sq_ref[...]
    p = jax.lax.dot_general(-2.0 * xr_ref[...], xc, dims,
                            preferred_element_type=jnp.float32)
    q = jax.lax.dot_general(-2.0 * xdr_ref[...], xc, dims,
                            preferred_element_type=jnp.float32)

    is_first = j == 0
    is_last = j == nc - 1

    for blk in range(R_TILE // R_SUB):
        i0 = blk * R_SUB
        kb = sq_c + p[i0:i0 + R_SUB, :]
        vb = sq_c + q[i0:i0 + R_SUB, :]
        tk, tv = _extract_top8(kb, vb)

        if nc == 1:
            nk, nv = tk, tv
        else:
            prev_k = jnp.where(is_first, BIG_F32, a_ref[i0:i0 + R_SUB, :])
            prev_v = jnp.where(is_first, 0.0, b_ref[i0:i0 + R_SUB, :])
            ck = jnp.concatenate([prev_k, tk], axis=1)
            cv = jnp.concatenate([prev_v, tv], axis=1)
            nk, nv = _extract_top8(ck, cv)

        fin_k = nk + jnp.sum(jnp.square(xr_ref[i0:i0 + R_SUB, :]),
                             axis=1, keepdims=True)
        fin_v = nv + jnp.sum(jnp.square(xdr_ref[i0:i0 + R_SUB, :]),
                             axis=1, keepdims=True)
        a_ref[i0:i0 + R_SUB, :] = jnp.where(is_last, fin_k, nk)
        b_ref[i0:i0 + R_SUB, :] = jnp.where(is_last, fin_v, nv)


def _finalize_body(a_ref, b_ref, o_ref):
    a = a_ref[...]
    b = b_ref[...]
    t = jnp.mean(a) + 1e-9
    o_ref[0, 0] = jnp.mean(jnp.exp(-a / t) * b)


@jax.jit
def kernel(X, X_dot):
    nr = N // R_TILE
    nc = N // C_TILE
    a, b = pl.pallas_call(
        functools.partial(_topk_body, nc),
        grid=(nr, nc),
        in_specs=[
            pl.BlockSpec((R_TILE, D), lambda i, j: (i, 0)),
            pl.BlockSpec((R_TILE, D), lambda i, j: (i, 0)),
            pl.BlockSpec((C_TILE, D), lambda i, j: (j, 0)),
        ],
        out_specs=[
            pl.BlockSpec((R_TILE, K), lambda i, j: (i, 0)),
            pl.BlockSpec((R_TILE, K), lambda i, j: (i, 0)),
        ],
        out_shape=[
            jax.ShapeDtypeStruct((N, K), jnp.float32),
            jax.ShapeDtypeStruct((N, K), jnp.float32),
        ],
        scratch_shapes=[pltpu.VMEM((1, C_TILE), jnp.float32)],
        compiler_params=pltpu.CompilerParams(
            dimension_semantics=("arbitrary", "arbitrary"),
        ),
    )(X, X_dot, X)

    out = pl.pallas_call(
        _finalize_body,
        out_specs=pl.BlockSpec(memory_space=pltpu.SMEM),
        out_shape=jax.ShapeDtypeStruct((1, 1), jnp.float32),
    )(a, b)
    return out[0, 0]
